# Initial kernel scaffold; baseline (speedup 1.0000x reference)
import jax
import jax.numpy as jnp
from jax.experimental import pallas as pl


def kernel(hidden_states, router_w, gate_up_proj, down_proj, shared_gate, shared_up, shared_down):
    raise NotImplementedError("write your pallas kernel here")


# speedup vs baseline: 2.9215x; 2.9215x over previous
import functools

import jax
import jax.numpy as jnp
from jax import lax
from jax.experimental import pallas as pl
from jax.experimental.pallas import tpu as pltpu
from jax.experimental.pallas import tpu_sc as plsc

E = 8
H = 1024
I = 2048
T = 2048
TILE = 256
NT = 16
P = NT * TILE
IB = 1024
F = I // IB
NC = 2
NS = 16
NW = NC * NS
TPW = T // NW


def _router_body(hs_ref, rw_ref, scores_ref, hss_ref, dest_ref, tid_ref, nv_ref):
    hs = hs_ref[...]
    logits = lax.dot_general(hs, rw_ref[...], (((1,), (1,)), ((), ())),
                             preferred_element_type=jnp.float32)
    iota_e = lax.broadcasted_iota(jnp.int32, (T, E), 1)
    mx = jnp.max(logits, axis=1, keepdims=True)
    eid = jnp.min(jnp.where(logits == mx, iota_e, E), axis=1, keepdims=True)
    onehot_b = iota_e == eid
    scores = jnp.where(onehot_b, jax.nn.sigmoid(logits), 0.0)
    scores_ref[...] = scores
    gval = jnp.sum(scores, axis=1, keepdims=True)
    hss_ref[...] = hs * gval

    oh = onehot_b.astype(jnp.float32)
    tt_r = lax.broadcasted_iota(jnp.int32, (T, T), 0)
    tt_c = lax.broadcasted_iota(jnp.int32, (T, T), 1)
    incl = (tt_c <= tt_r).astype(jnp.float32)
    cum = jnp.dot(incl, oh, preferred_element_type=jnp.float32)

    counts = cum[T - 1:T, :]
    padded = jnp.floor((counts + (TILE - 1)) / TILE) * TILE
    ee_r = lax.broadcasted_iota(jnp.int32, (E, E), 0)
    ee_c = lax.broadcasted_iota(jnp.int32, (E, E), 1)
    lt = (ee_r < ee_c).astype(jnp.float32)
    le = (ee_r <= ee_c).astype(jnp.float32)
    start = jnp.dot(padded, lt, preferred_element_type=jnp.float32)
    bound = jnp.dot(padded, le, preferred_element_type=jnp.float32)
    total = bound[:, E - 1:E]

    dest = jnp.sum(oh * (start + cum - 1.0), axis=1, keepdims=True)
    dest_ref[...] = dest.astype(jnp.int32)

    ti = lax.broadcasted_iota(jnp.int32, (NT, 1), 0).astype(jnp.float32) * TILE
    tpos = jnp.minimum(ti, total - TILE)
    tid = jnp.sum((bound <= tpos).astype(jnp.float32), axis=1, keepdims=True)
    tid_ref[...] = tid.astype(jnp.int32)
    nv_ref[...] = (total / TILE).astype(jnp.int32)


def _router_call(hs, router_w, interpret=False):
    return pl.pallas_call(
        _router_body,
        out_shape=[
            jax.ShapeDtypeStruct((T, E), jnp.float32),
            jax.ShapeDtypeStruct((T, H), jnp.float32),
            jax.ShapeDtypeStruct((T, 1), jnp.int32),
            jax.ShapeDtypeStruct((NT, 1), jnp.int32),
            jax.ShapeDtypeStruct((1, 1), jnp.int32),
        ],
        interpret=interpret,
    )(hs, router_w)


def _expert_body(meta_ref, x_ref, g_ref, u_ref, d_ref, o_ref):
    r = pl.program_id(0)
    f = pl.program_id(1)
    nv = meta_ref[NT]

    @pl.when(r < nv)
    def _():
        x = x_ref[...]
        g = jnp.dot(x, g_ref[0], preferred_element_type=jnp.float32)
        u = jnp.dot(x, u_ref[0], preferred_element_type=jnp.float32)
        h = (g * jax.nn.sigmoid(g)) * u
        part = jnp.dot(h, d_ref[0], preferred_element_type=jnp.float32)

        @pl.when(f == 0)
        def _():
            o_ref[...] = part

        @pl.when(f != 0)
        def _():
            o_ref[...] += part


def _xm(r, f, meta):
    nv = meta[NT]
    return (jnp.where(r < nv, r, nv - 1), 0)


def _gm(r, f, meta):
    nv = meta[NT]
    return (meta[r], 0, jnp.where(r < nv, f, F - 1))


def _um(r, f, meta):
    nv = meta[NT]
    return (meta[r], 0, F + jnp.where(r < nv, f, F - 1))


def _dm(r, f, meta):
    nv = meta[NT]
    return (meta[r], jnp.where(r < nv, f, F - 1), 0)


def _om(r, f, meta):
    nv = meta[NT]
    return (jnp.where(r < nv, r, nv - 1), 0)


def _expert_call(meta, xbuf, gate_up_proj, down_proj, interpret=False):
    grid_spec = pltpu.PrefetchScalarGridSpec(
        num_scalar_prefetch=1,
        grid=(NT, F),
        in_specs=[
            pl.BlockSpec((TILE, H), _xm),
            pl.BlockSpec((1, H, IB), _gm),
            pl.BlockSpec((1, H, IB), _um),
            pl.BlockSpec((1, IB, H), _dm),
        ],
        out_specs=pl.BlockSpec((TILE, H), _om),
    )
    return pl.pallas_call(
        _expert_body,
        grid_spec=grid_spec,
        out_shape=jax.ShapeDtypeStruct((P, H), jnp.float32),
        compiler_params=pltpu.CompilerParams(
            dimension_semantics=("arbitrary", "arbitrary")),
        interpret=interpret,
    )(meta, xbuf, gate_up_proj, gate_up_proj, down_proj)


def _shared_body(hs_ref, sg_ref, su_ref, sd_ref, rt_ref, o_ref):
    f = pl.program_id(1)
    x = hs_ref[...]
    g = jnp.dot(x, sg_ref[...], preferred_element_type=jnp.float32)
    u = jnp.dot(x, su_ref[...], preferred_element_type=jnp.float32)
    h = (g * jax.nn.sigmoid(g)) * u
    part = jnp.dot(h, sd_ref[...], preferred_element_type=jnp.float32)

    @pl.when(f == 0)
    def _():
        o_ref[...] = part + rt_ref[...]

    @pl.when(f != 0)
    def _():
        o_ref[...] += part


def _shared_call(hs, shared_gate, shared_up, shared_down, rt, interpret=False):
    return pl.pallas_call(
        _shared_body,
        grid=(T // TILE, F),
        in_specs=[
            pl.BlockSpec((TILE, H), lambda r, f: (r, 0)),
            pl.BlockSpec((H, IB), lambda r, f: (0, f)),
            pl.BlockSpec((H, IB), lambda r, f: (0, f)),
            pl.BlockSpec((IB, H), lambda r, f: (f, 0)),
            pl.BlockSpec((TILE, H), lambda r, f: (r, 0)),
        ],
        out_specs=pl.BlockSpec((TILE, H), lambda r, f: (r, 0)),
        out_shape=jax.ShapeDtypeStruct((T, H), jnp.float32),
        compiler_params=pltpu.CompilerParams(
            dimension_semantics=("arbitrary", "arbitrary")),
        interpret=interpret,
    )(hs, shared_gate, shared_up, shared_down, rt)


_SC_MESH = dict(core_axis_name="c", subcore_axis_name="s")


def _sc_dispatch_body(hss_hbm, dest_hbm, xbuf_hbm, idx_v, rows_v, sem):
    wid = lax.axis_index("s") * NC + lax.axis_index("c")
    base = wid * TPW
    pltpu.sync_copy(dest_hbm.at[pl.ds(base, TPW)], idx_v)
    pltpu.sync_copy(hss_hbm.at[pl.ds(base, TPW)], rows_v)
    pltpu.async_copy(rows_v, xbuf_hbm.at[idx_v], sem).wait()


def _sc_dispatch(hss, dest):
    return pl.kernel(
        _sc_dispatch_body,
        out_type=jax.ShapeDtypeStruct((P, H), jnp.float32),
        mesh=plsc.VectorSubcoreMesh(**_SC_MESH),
        scratch_types=[
            pltpu.VMEM((TPW,), jnp.int32),
            pltpu.VMEM((TPW, H), jnp.float32),
            pltpu.SemaphoreType.DMA,
        ],
    )(hss, dest)


def _sc_gather_body(routed_hbm, dest_hbm, out_hbm, idx_v, rows_v, sem):
    wid = lax.axis_index("s") * NC + lax.axis_index("c")
    base = wid * TPW
    pltpu.sync_copy(dest_hbm.at[pl.ds(base, TPW)], idx_v)
    pltpu.async_copy(routed_hbm.at[idx_v], rows_v, sem).wait()
    pltpu.sync_copy(rows_v, out_hbm.at[pl.ds(base, TPW)])


def _sc_gather(routed, dest):
    return pl.kernel(
        _sc_gather_body,
        out_type=jax.ShapeDtypeStruct((T, H), jnp.float32),
        mesh=plsc.VectorSubcoreMesh(**_SC_MESH),
        scratch_types=[
            pltpu.VMEM((TPW,), jnp.int32),
            pltpu.VMEM((TPW, H), jnp.float32),
            pltpu.SemaphoreType.DMA,
        ],
    )(routed, dest)


def kernel(hidden_states, router_w, gate_up_proj, down_proj,
           shared_gate, shared_up, shared_down):
    hs = hidden_states.reshape(T, H)
    scores_te, hs_scaled, dest2, tid, nv = _router_call(hs, router_w)
    router_scores = scores_te.T
    dest = dest2.reshape(T)
    meta = jnp.concatenate([tid.reshape(NT), nv.reshape(1)])
    xbuf = _sc_dispatch(hs_scaled, dest)
    routed = _expert_call(meta, xbuf, gate_up_proj, down_proj)
    rt = _sc_gather(routed, dest)
    out = _shared_call(hs, shared_gate, shared_up, shared_down, rt)
    return out, router_scores


# speedup vs baseline: 2.9228x; 1.0005x over previous
import functools

import jax
import jax.numpy as jnp
from jax import lax
from jax.experimental import pallas as pl
from jax.experimental.pallas import tpu as pltpu
from jax.experimental.pallas import tpu_sc as plsc

E = 8
H = 1024
I = 2048
T = 2048
TILE = 256
NT = 16
P = NT * TILE
IB = 1024
F = I // IB
NC = 2
NS = 16
NW = NC * NS
TPW = T // NW


def _router_body(hs_ref, rw_ref, scores_ref, hss_ref, dest_ref, tid_ref, nv_ref):
    hs = hs_ref[...]
    logits = lax.dot_general(hs, rw_ref[...], (((1,), (1,)), ((), ())),
                             preferred_element_type=jnp.float32)
    iota_e = lax.broadcasted_iota(jnp.int32, (T, E), 1)
    mx = jnp.max(logits, axis=1, keepdims=True)
    eid = jnp.min(jnp.where(logits == mx, iota_e, E), axis=1, keepdims=True)
    onehot_b = iota_e == eid
    scores = jnp.where(onehot_b, jax.nn.sigmoid(logits), 0.0)
    scores_ref[...] = scores
    gval = jnp.sum(scores, axis=1, keepdims=True)
    hss_ref[...] = hs * gval

    oh = onehot_b.astype(jnp.float32)
    tt_r = lax.broadcasted_iota(jnp.int32, (T, T), 0)
    tt_c = lax.broadcasted_iota(jnp.int32, (T, T), 1)
    incl = (tt_c <= tt_r).astype(jnp.float32)
    cum = jnp.dot(incl, oh, preferred_element_type=jnp.float32)

    counts = cum[T - 1:T, :]
    padded = jnp.floor((counts + (TILE - 1)) / TILE) * TILE
    ee_r = lax.broadcasted_iota(jnp.int32, (E, E), 0)
    ee_c = lax.broadcasted_iota(jnp.int32, (E, E), 1)
    lt = (ee_r < ee_c).astype(jnp.float32)
    le = (ee_r <= ee_c).astype(jnp.float32)
    start = jnp.dot(padded, lt, preferred_element_type=jnp.float32)
    bound = jnp.dot(padded, le, preferred_element_type=jnp.float32)
    total = bound[:, E - 1:E]

    dest = jnp.sum(oh * (start + cum - 1.0), axis=1, keepdims=True)
    dest_ref[...] = dest.astype(jnp.int32)

    ti = lax.broadcasted_iota(jnp.int32, (NT, 1), 0).astype(jnp.float32) * TILE
    tpos = jnp.minimum(ti, total - TILE)
    tid = jnp.sum((bound <= tpos).astype(jnp.float32), axis=1, keepdims=True)
    tid_ref[...] = tid.astype(jnp.int32)
    nv_ref[...] = (total / TILE).astype(jnp.int32)


def _router_call(hs, router_w, interpret=False):
    return pl.pallas_call(
        _router_body,
        out_shape=[
            jax.ShapeDtypeStruct((T, E), jnp.float32),
            jax.ShapeDtypeStruct((T, H), jnp.float32),
            jax.ShapeDtypeStruct((T, 1), jnp.int32),
            jax.ShapeDtypeStruct((NT, 1), jnp.int32),
            jax.ShapeDtypeStruct((1, 1), jnp.int32),
        ],
        interpret=interpret,
    )(hs, router_w)


def _expert_body(meta_ref, x_ref, g_ref, u_ref, d_ref, o_ref):
    r = pl.program_id(0)
    f = pl.program_id(1)
    nv = meta_ref[NT]

    @pl.when(r < nv)
    def _():
        x = x_ref[...].astype(jnp.bfloat16)
        g = jnp.dot(x, g_ref[0].astype(jnp.bfloat16),
                    preferred_element_type=jnp.float32)
        u = jnp.dot(x, u_ref[0].astype(jnp.bfloat16),
                    preferred_element_type=jnp.float32)
        h = ((g * jax.nn.sigmoid(g)) * u).astype(jnp.bfloat16)
        part = jnp.dot(h, d_ref[0].astype(jnp.bfloat16),
                       preferred_element_type=jnp.float32)

        @pl.when(f == 0)
        def _():
            o_ref[...] = part

        @pl.when(f != 0)
        def _():
            o_ref[...] += part


def _xm(r, f, meta):
    nv = meta[NT]
    return (jnp.where(r < nv, r, nv - 1), 0)


def _gm(r, f, meta):
    nv = meta[NT]
    return (meta[r], 0, jnp.where(r < nv, f, F - 1))


def _um(r, f, meta):
    nv = meta[NT]
    return (meta[r], 0, F + jnp.where(r < nv, f, F - 1))


def _dm(r, f, meta):
    nv = meta[NT]
    return (meta[r], jnp.where(r < nv, f, F - 1), 0)


def _om(r, f, meta):
    nv = meta[NT]
    return (jnp.where(r < nv, r, nv - 1), 0)


def _expert_call(meta, xbuf, gate_up_proj, down_proj, interpret=False):
    grid_spec = pltpu.PrefetchScalarGridSpec(
        num_scalar_prefetch=1,
        grid=(NT, F),
        in_specs=[
            pl.BlockSpec((TILE, H), _xm),
            pl.BlockSpec((1, H, IB), _gm),
            pl.BlockSpec((1, H, IB), _um),
            pl.BlockSpec((1, IB, H), _dm),
        ],
        out_specs=pl.BlockSpec((TILE, H), _om),
    )
    return pl.pallas_call(
        _expert_body,
        grid_spec=grid_spec,
        out_shape=jax.ShapeDtypeStruct((P, H), jnp.float32),
        compiler_params=pltpu.CompilerParams(
            dimension_semantics=("arbitrary", "arbitrary")),
        interpret=interpret,
    )(meta, xbuf, gate_up_proj, gate_up_proj, down_proj)


def _shared_body(hs_ref, sg_ref, su_ref, sd_ref, rt_ref, o_ref):
    f = pl.program_id(1)
    x = hs_ref[...].astype(jnp.bfloat16)
    g = jnp.dot(x, sg_ref[...].astype(jnp.bfloat16),
                preferred_element_type=jnp.float32)
    u = jnp.dot(x, su_ref[...].astype(jnp.bfloat16),
                preferred_element_type=jnp.float32)
    h = ((g * jax.nn.sigmoid(g)) * u).astype(jnp.bfloat16)
    part = jnp.dot(h, sd_ref[...].astype(jnp.bfloat16),
                   preferred_element_type=jnp.float32)

    @pl.when(f == 0)
    def _():
        o_ref[...] = part + rt_ref[...]

    @pl.when(f != 0)
    def _():
        o_ref[...] += part


def _shared_call(hs, shared_gate, shared_up, shared_down, rt, interpret=False):
    return pl.pallas_call(
        _shared_body,
        grid=(T // TILE, F),
        in_specs=[
            pl.BlockSpec((TILE, H), lambda r, f: (r, 0)),
            pl.BlockSpec((H, IB), lambda r, f: (0, f)),
            pl.BlockSpec((H, IB), lambda r, f: (0, f)),
            pl.BlockSpec((IB, H), lambda r, f: (f, 0)),
            pl.BlockSpec((TILE, H), lambda r, f: (r, 0)),
        ],
        out_specs=pl.BlockSpec((TILE, H), lambda r, f: (r, 0)),
        out_shape=jax.ShapeDtypeStruct((T, H), jnp.float32),
        compiler_params=pltpu.CompilerParams(
            dimension_semantics=("arbitrary", "arbitrary")),
        interpret=interpret,
    )(hs, shared_gate, shared_up, shared_down, rt)


_SC_MESH = dict(core_axis_name="c", subcore_axis_name="s")


def _sc_dispatch_body(hss_hbm, dest_hbm, xbuf_hbm, idx_v, rows_v, sem):
    wid = lax.axis_index("s") * NC + lax.axis_index("c")
    base = wid * TPW
    pltpu.sync_copy(dest_hbm.at[pl.ds(base, TPW)], idx_v)
    pltpu.sync_copy(hss_hbm.at[pl.ds(base, TPW)], rows_v)
    pltpu.async_copy(rows_v, xbuf_hbm.at[idx_v], sem).wait()


def _sc_dispatch(hss, dest):
    return pl.kernel(
        _sc_dispatch_body,
        out_type=jax.ShapeDtypeStruct((P, H), jnp.float32),
        mesh=plsc.VectorSubcoreMesh(**_SC_MESH),
        scratch_types=[
            pltpu.VMEM((TPW,), jnp.int32),
            pltpu.VMEM((TPW, H), jnp.float32),
            pltpu.SemaphoreType.DMA,
        ],
    )(hss, dest)


def _sc_gather_body(routed_hbm, dest_hbm, out_hbm, idx_v, rows_v, sem):
    wid = lax.axis_index("s") * NC + lax.axis_index("c")
    base = wid * TPW
    pltpu.sync_copy(dest_hbm.at[pl.ds(base, TPW)], idx_v)
    pltpu.async_copy(routed_hbm.at[idx_v], rows_v, sem).wait()
    pltpu.sync_copy(rows_v, out_hbm.at[pl.ds(base, TPW)])


def _sc_gather(routed, dest):
    return pl.kernel(
        _sc_gather_body,
        out_type=jax.ShapeDtypeStruct((T, H), jnp.float32),
        mesh=plsc.VectorSubcoreMesh(**_SC_MESH),
        scratch_types=[
            pltpu.VMEM((TPW,), jnp.int32),
            pltpu.VMEM((TPW, H), jnp.float32),
            pltpu.SemaphoreType.DMA,
        ],
    )(routed, dest)


def kernel(hidden_states, router_w, gate_up_proj, down_proj,
           shared_gate, shared_up, shared_down):
    hs = hidden_states.reshape(T, H)
    scores_te, hs_scaled, dest2, tid, nv = _router_call(hs, router_w)
    router_scores = scores_te.T
    dest = dest2.reshape(T)
    meta = jnp.concatenate([tid.reshape(NT), nv.reshape(1)])
    xbuf = _sc_dispatch(hs_scaled, dest)
    routed = _expert_call(meta, xbuf, gate_up_proj, down_proj)
    rt = _sc_gather(routed, dest)
    out = _shared_call(hs, shared_gate, shared_up, shared_down, rt)
    return out, router_scores


# speedup vs baseline: 3.2841x; 1.1236x over previous
import functools

import jax
import jax.numpy as jnp
from jax import lax
from jax.experimental import pallas as pl
from jax.experimental.pallas import tpu as pltpu
from jax.experimental.pallas import tpu_sc as plsc

E = 8
H = 1024
I = 2048
T = 2048
TILE = 128
NT = 24
P = NT * TILE
STILE = 256
NC = 2
NS = 16
NW = NC * NS
TPW = T // NW


def _router_body(hs_ref, rw_ref, scores_ref, hss_ref, dest_ref, tid_ref, nv_ref):
    hs = hs_ref[...]
    logits = lax.dot_general(hs, rw_ref[...], (((1,), (1,)), ((), ())),
                             preferred_element_type=jnp.float32)
    iota_e = lax.broadcasted_iota(jnp.int32, (T, E), 1)
    mx = jnp.max(logits, axis=1, keepdims=True)
    eid = jnp.min(jnp.where(logits == mx, iota_e, E), axis=1, keepdims=True)
    onehot_b = iota_e == eid
    scores = jnp.where(onehot_b, jax.nn.sigmoid(logits), 0.0)
    scores_ref[...] = scores
    gval = jnp.sum(scores, axis=1, keepdims=True)
    hss_ref[...] = hs * gval

    oh = onehot_b.astype(jnp.float32)
    tt_r = lax.broadcasted_iota(jnp.int32, (T, T), 0)
    tt_c = lax.broadcasted_iota(jnp.int32, (T, T), 1)
    incl = (tt_c <= tt_r).astype(jnp.float32)
    cum = jnp.dot(incl, oh, preferred_element_type=jnp.float32)

    counts = cum[T - 1:T, :]
    padded = jnp.floor((counts + (TILE - 1)) / TILE) * TILE
    ee_r = lax.broadcasted_iota(jnp.int32, (E, E), 0)
    ee_c = lax.broadcasted_iota(jnp.int32, (E, E), 1)
    lt = (ee_r < ee_c).astype(jnp.float32)
    le = (ee_r <= ee_c).astype(jnp.float32)
    start = jnp.dot(padded, lt, preferred_element_type=jnp.float32)
    bound = jnp.dot(padded, le, preferred_element_type=jnp.float32)
    total = bound[:, E - 1:E]

    dest = jnp.sum(oh * (start + cum - 1.0), axis=1, keepdims=True)
    dest_ref[...] = dest.astype(jnp.int32)

    ti = lax.broadcasted_iota(jnp.int32, (NT, 1), 0).astype(jnp.float32) * TILE
    tpos = jnp.minimum(ti, total - TILE)
    tid = jnp.sum((bound <= tpos).astype(jnp.float32), axis=1, keepdims=True)
    tid_ref[...] = tid.astype(jnp.int32)
    nv_ref[...] = (total / TILE).astype(jnp.int32)


def _router_call(hs, router_w, interpret=False):
    return pl.pallas_call(
        _router_body,
        out_shape=[
            jax.ShapeDtypeStruct((T, E), jnp.float32),
            jax.ShapeDtypeStruct((T, H), jnp.float32),
            jax.ShapeDtypeStruct((T, 1), jnp.int32),
            jax.ShapeDtypeStruct((NT, 1), jnp.int32),
            jax.ShapeDtypeStruct((1, 1), jnp.int32),
        ],
        interpret=interpret,
    )(hs, router_w)


def _expert_body(meta_ref, x_ref, g_ref, u_ref, d_ref, o_ref):
    r = pl.program_id(0)
    nv = meta_ref[NT]

    @pl.when(r < nv)
    def _():
        x = x_ref[...].astype(jnp.bfloat16)
        g = jnp.dot(x, g_ref[0].astype(jnp.bfloat16),
                    preferred_element_type=jnp.float32)
        u = jnp.dot(x, u_ref[0].astype(jnp.bfloat16),
                    preferred_element_type=jnp.float32)
        h = ((g * jax.nn.sigmoid(g)) * u).astype(jnp.bfloat16)
        o_ref[...] = jnp.dot(h, d_ref[0].astype(jnp.bfloat16),
                             preferred_element_type=jnp.float32)


def _xm(r, meta):
    nv = meta[NT]
    return (jnp.where(r < nv, r, nv - 1), 0)


def _gm(r, meta):
    return (meta[r], 0, 0)


def _um(r, meta):
    return (meta[r], 0, 1)


def _dm(r, meta):
    return (meta[r], 0, 0)


def _expert_call(meta, xbuf, gate_up_proj, down_proj, interpret=False):
    grid_spec = pltpu.PrefetchScalarGridSpec(
        num_scalar_prefetch=1,
        grid=(NT,),
        in_specs=[
            pl.BlockSpec((TILE, H), _xm),
            pl.BlockSpec((1, H, I), _gm),
            pl.BlockSpec((1, H, I), _um),
            pl.BlockSpec((1, I, H), _dm),
        ],
        out_specs=pl.BlockSpec((TILE, H), _xm),
    )
    return pl.pallas_call(
        _expert_body,
        grid_spec=grid_spec,
        out_shape=jax.ShapeDtypeStruct((P, H), jnp.float32),
        compiler_params=pltpu.CompilerParams(
            dimension_semantics=("arbitrary",)),
        interpret=interpret,
    )(meta, xbuf, gate_up_proj, gate_up_proj, down_proj)


def _shared_body(hs_ref, sg_ref, su_ref, sd_ref, rt_ref, o_ref):
    x = hs_ref[...].astype(jnp.bfloat16)
    g = jnp.dot(x, sg_ref[...], preferred_element_type=jnp.float32)
    u = jnp.dot(x, su_ref[...], preferred_element_type=jnp.float32)
    h = ((g * jax.nn.sigmoid(g)) * u).astype(jnp.bfloat16)
    o_ref[...] = jnp.dot(h, sd_ref[...],
                         preferred_element_type=jnp.float32) + rt_ref[...]


def _shared_call(hs, sg_bf, su_bf, sd_bf, rt, interpret=False):
    return pl.pallas_call(
        _shared_body,
        grid=(T // STILE,),
        in_specs=[
            pl.BlockSpec((STILE, H), lambda r: (r, 0)),
            pl.BlockSpec((H, I), lambda r: (0, 0)),
            pl.BlockSpec((H, I), lambda r: (0, 0)),
            pl.BlockSpec((I, H), lambda r: (0, 0)),
            pl.BlockSpec((STILE, H), lambda r: (r, 0)),
        ],
        out_specs=pl.BlockSpec((STILE, H), lambda r: (r, 0)),
        out_shape=jax.ShapeDtypeStruct((T, H), jnp.float32),
        compiler_params=pltpu.CompilerParams(
            dimension_semantics=("arbitrary",)),
        interpret=interpret,
    )(hs, sg_bf, su_bf, sd_bf, rt)


_SC_MESH = dict(core_axis_name="c", subcore_axis_name="s")


def _sc_dispatch_body(hss_hbm, dest_hbm, xbuf_hbm, idx_v, rows_v, sem):
    wid = lax.axis_index("s") * NC + lax.axis_index("c")
    base = wid * TPW
    pltpu.sync_copy(dest_hbm.at[pl.ds(base, TPW)], idx_v)
    pltpu.sync_copy(hss_hbm.at[pl.ds(base, TPW)], rows_v)
    pltpu.async_copy(rows_v, xbuf_hbm.at[idx_v], sem).wait()


def _sc_dispatch(hss, dest):
    return pl.kernel(
        _sc_dispatch_body,
        out_type=jax.ShapeDtypeStruct((P, H), jnp.float32),
        mesh=plsc.VectorSubcoreMesh(**_SC_MESH),
        scratch_types=[
            pltpu.VMEM((TPW,), jnp.int32),
            pltpu.VMEM((TPW, H), jnp.float32),
            pltpu.SemaphoreType.DMA,
        ],
    )(hss, dest)


def _sc_gather_body(routed_hbm, dest_hbm, out_hbm, idx_v, rows_v, sem):
    wid = lax.axis_index("s") * NC + lax.axis_index("c")
    base = wid * TPW
    pltpu.sync_copy(dest_hbm.at[pl.ds(base, TPW)], idx_v)
    pltpu.async_copy(routed_hbm.at[idx_v], rows_v, sem).wait()
    pltpu.sync_copy(rows_v, out_hbm.at[pl.ds(base, TPW)])


def _sc_gather(routed, dest):
    return pl.kernel(
        _sc_gather_body,
        out_type=jax.ShapeDtypeStruct((T, H), jnp.float32),
        mesh=plsc.VectorSubcoreMesh(**_SC_MESH),
        scratch_types=[
            pltpu.VMEM((TPW,), jnp.int32),
            pltpu.VMEM((TPW, H), jnp.float32),
            pltpu.SemaphoreType.DMA,
        ],
    )(routed, dest)


def kernel(hidden_states, router_w, gate_up_proj, down_proj,
           shared_gate, shared_up, shared_down):
    hs = hidden_states.reshape(T, H)
    scores_te, hs_scaled, dest2, tid, nv = _router_call(hs, router_w)
    router_scores = scores_te.T
    dest = dest2.reshape(T)
    meta = jnp.concatenate([tid.reshape(NT), nv.reshape(1)])
    xbuf = _sc_dispatch(hs_scaled, dest)
    routed = _expert_call(meta, xbuf, gate_up_proj, down_proj)
    rt = _sc_gather(routed, dest)
    out = _shared_call(hs, shared_gate.astype(jnp.bfloat16),
                       shared_up.astype(jnp.bfloat16),
                       shared_down.astype(jnp.bfloat16), rt)
    return out, router_scores


# speedup vs baseline: 3.4298x; 1.0444x over previous
import functools

import jax
import jax.numpy as jnp
from jax import lax
from jax.experimental import pallas as pl
from jax.experimental.pallas import tpu as pltpu
from jax.experimental.pallas import tpu_sc as plsc

E = 8
H = 1024
I = 2048
T = 2048
TILE = 128
NT = 24
P = NT * TILE
STILE = 256
NC = 2
NS = 16
NW = NC * NS
TPW = T // NW


def _router_body(hs_ref, rw_ref, scores_ref, hss_ref, dest_ref, tid_ref, nv_ref):
    hs = hs_ref[...]
    logits = lax.dot_general(hs, rw_ref[...], (((1,), (1,)), ((), ())),
                             preferred_element_type=jnp.float32)
    iota_e = lax.broadcasted_iota(jnp.int32, (T, E), 1)
    mx = jnp.max(logits, axis=1, keepdims=True)
    eid = jnp.min(jnp.where(logits == mx, iota_e, E), axis=1, keepdims=True)
    onehot_b = iota_e == eid
    scores = jnp.where(onehot_b, jax.nn.sigmoid(logits), 0.0)
    scores_ref[...] = scores
    gval = jnp.sum(scores, axis=1, keepdims=True)
    hss_ref[...] = hs * gval

    oh = onehot_b.astype(jnp.bfloat16)
    tt_r = lax.broadcasted_iota(jnp.int32, (T, T), 0)
    tt_c = lax.broadcasted_iota(jnp.int32, (T, T), 1)
    incl = (tt_c <= tt_r).astype(jnp.bfloat16)
    cum = jnp.dot(incl, oh, preferred_element_type=jnp.float32)

    counts = cum[T - 1:T, :]
    padded = jnp.floor((counts + (TILE - 1)) / TILE) * TILE
    ee_r = lax.broadcasted_iota(jnp.int32, (E, E), 0)
    ee_c = lax.broadcasted_iota(jnp.int32, (E, E), 1)
    lt = (ee_r < ee_c).astype(jnp.float32)
    le = (ee_r <= ee_c).astype(jnp.float32)
    start = jnp.dot(padded, lt, preferred_element_type=jnp.float32)
    bound = jnp.dot(padded, le, preferred_element_type=jnp.float32)
    total = bound[:, E - 1:E]

    dest = jnp.sum(oh * (start + cum - 1.0), axis=1, keepdims=True)
    dest_ref[...] = dest.astype(jnp.int32)

    ti = lax.broadcasted_iota(jnp.int32, (NT, 1), 0).astype(jnp.float32) * TILE
    tpos = jnp.minimum(ti, total - TILE)
    tid = jnp.sum((bound <= tpos).astype(jnp.float32), axis=1, keepdims=True)
    tid_ref[...] = tid.astype(jnp.int32)
    nv_ref[...] = (total / TILE).astype(jnp.int32)


def _router_call(hs, router_w, interpret=False):
    return pl.pallas_call(
        _router_body,
        out_shape=[
            jax.ShapeDtypeStruct((T, E), jnp.float32),
            jax.ShapeDtypeStruct((T, H), jnp.float32),
            jax.ShapeDtypeStruct((T, 1), jnp.int32),
            jax.ShapeDtypeStruct((NT, 1), jnp.int32),
            jax.ShapeDtypeStruct((1, 1), jnp.int32),
        ],
        interpret=interpret,
    )(hs, router_w)


def _expert_body(meta_ref, x_ref, g_ref, u_ref, d_ref, o_ref):
    r = pl.program_id(0)
    nv = meta_ref[NT]

    @pl.when(r < nv)
    def _():
        x = x_ref[...].astype(jnp.bfloat16)
        g = jnp.dot(x, g_ref[0].astype(jnp.bfloat16),
                    preferred_element_type=jnp.float32)
        u = jnp.dot(x, u_ref[0].astype(jnp.bfloat16),
                    preferred_element_type=jnp.float32)
        h = ((g * jax.nn.sigmoid(g)) * u).astype(jnp.bfloat16)
        o_ref[...] = jnp.dot(h, d_ref[0].astype(jnp.bfloat16),
                             preferred_element_type=jnp.float32)


def _xm(r, meta):
    nv = meta[NT]
    return (jnp.where(r < nv, r, nv - 1), 0)


def _gm(r, meta):
    return (meta[r], 0, 0)


def _um(r, meta):
    return (meta[r], 0, 1)


def _dm(r, meta):
    return (meta[r], 0, 0)


def _expert_call(meta, xbuf, gate_up_proj, down_proj, interpret=False):
    grid_spec = pltpu.PrefetchScalarGridSpec(
        num_scalar_prefetch=1,
        grid=(NT,),
        in_specs=[
            pl.BlockSpec((TILE, H), _xm),
            pl.BlockSpec((1, H, I), _gm),
            pl.BlockSpec((1, H, I), _um),
            pl.BlockSpec((1, I, H), _dm),
        ],
        out_specs=pl.BlockSpec((TILE, H), _xm),
    )
    return pl.pallas_call(
        _expert_body,
        grid_spec=grid_spec,
        out_shape=jax.ShapeDtypeStruct((P, H), jnp.float32),
        compiler_params=pltpu.CompilerParams(
            dimension_semantics=("arbitrary",)),
        interpret=interpret,
    )(meta, xbuf, gate_up_proj, gate_up_proj, down_proj)


def _shared_body(hs_ref, sg_ref, su_ref, sd_ref, rt_ref, o_ref,
                 sgs_ref, sus_ref, sds_ref):
    r = pl.program_id(0)

    @pl.when(r == 0)
    def _():
        sgs_ref[...] = sg_ref[...].astype(jnp.bfloat16)
        sus_ref[...] = su_ref[...].astype(jnp.bfloat16)
        sds_ref[...] = sd_ref[...].astype(jnp.bfloat16)

    x = hs_ref[...].astype(jnp.bfloat16)
    g = jnp.dot(x, sgs_ref[...], preferred_element_type=jnp.float32)
    u = jnp.dot(x, sus_ref[...], preferred_element_type=jnp.float32)
    h = ((g * jax.nn.sigmoid(g)) * u).astype(jnp.bfloat16)
    o_ref[...] = jnp.dot(h, sds_ref[...],
                         preferred_element_type=jnp.float32) + rt_ref[...]


def _shared_call(hs, shared_gate, shared_up, shared_down, rt, interpret=False):
    return pl.pallas_call(
        _shared_body,
        grid=(T // STILE,),
        in_specs=[
            pl.BlockSpec((STILE, H), lambda r: (r, 0)),
            pl.BlockSpec((H, I), lambda r: (0, 0)),
            pl.BlockSpec((H, I), lambda r: (0, 0)),
            pl.BlockSpec((I, H), lambda r: (0, 0)),
            pl.BlockSpec((STILE, H), lambda r: (r, 0)),
        ],
        out_specs=pl.BlockSpec((STILE, H), lambda r: (r, 0)),
        out_shape=jax.ShapeDtypeStruct((T, H), jnp.float32),
        scratch_shapes=[
            pltpu.VMEM((H, I), jnp.bfloat16),
            pltpu.VMEM((H, I), jnp.bfloat16),
            pltpu.VMEM((I, H), jnp.bfloat16),
        ],
        compiler_params=pltpu.CompilerParams(
            dimension_semantics=("arbitrary",)),
        interpret=interpret,
    )(hs, shared_gate, shared_up, shared_down, rt)


_SC_MESH = dict(core_axis_name="c", subcore_axis_name="s")


def _sc_dispatch_body(hss_hbm, dest_hbm, xbuf_hbm, idx_v, rows_v, sem):
    wid = lax.axis_index("s") * NC + lax.axis_index("c")
    base = wid * TPW
    pltpu.sync_copy(dest_hbm.at[pl.ds(base, TPW)], idx_v)
    pltpu.sync_copy(hss_hbm.at[pl.ds(base, TPW)], rows_v)
    pltpu.async_copy(rows_v, xbuf_hbm.at[idx_v], sem).wait()


def _sc_dispatch(hss, dest):
    return pl.kernel(
        _sc_dispatch_body,
        out_type=jax.ShapeDtypeStruct((P, H), jnp.float32),
        mesh=plsc.VectorSubcoreMesh(**_SC_MESH),
        scratch_types=[
            pltpu.VMEM((TPW,), jnp.int32),
            pltpu.VMEM((TPW, H), jnp.float32),
            pltpu.SemaphoreType.DMA,
        ],
    )(hss, dest)


def _sc_gather_body(routed_hbm, dest_hbm, out_hbm, idx_v, rows_v, sem):
    wid = lax.axis_index("s") * NC + lax.axis_index("c")
    base = wid * TPW
    pltpu.sync_copy(dest_hbm.at[pl.ds(base, TPW)], idx_v)
    pltpu.async_copy(routed_hbm.at[idx_v], rows_v, sem).wait()
    pltpu.sync_copy(rows_v, out_hbm.at[pl.ds(base, TPW)])


def _sc_gather(routed, dest):
    return pl.kernel(
        _sc_gather_body,
        out_type=jax.ShapeDtypeStruct((T, H), jnp.float32),
        mesh=plsc.VectorSubcoreMesh(**_SC_MESH),
        scratch_types=[
            pltpu.VMEM((TPW,), jnp.int32),
            pltpu.VMEM((TPW, H), jnp.float32),
            pltpu.SemaphoreType.DMA,
        ],
    )(routed, dest)


def kernel(hidden_states, router_w, gate_up_proj, down_proj,
           shared_gate, shared_up, shared_down):
    hs = hidden_states.reshape(T, H)
    scores_te, hs_scaled, dest2, tid, nv = _router_call(hs, router_w)
    router_scores = scores_te.T
    dest = dest2.reshape(T)
    meta = jnp.concatenate([tid.reshape(NT), nv.reshape(1)])
    xbuf = _sc_dispatch(hs_scaled, dest)
    routed = _expert_call(meta, xbuf, gate_up_proj, down_proj)
    rt = _sc_gather(routed, dest)
    out = _shared_call(hs, shared_gate, shared_up, shared_down, rt)
    return out, router_scores


# speedup vs baseline: 3.4402x; 1.0030x over previous
import functools

import jax
import jax.numpy as jnp
from jax import lax
from jax.experimental import pallas as pl
from jax.experimental.pallas import tpu as pltpu
from jax.experimental.pallas import tpu_sc as plsc

E = 8
H = 1024
I = 2048
T = 2048
TILE = 128
NT = 24
P = NT * TILE
STILE = 512
NC = 2
NS = 16
NW = NC * NS
TPW = T // NW


def _router_body(hs_ref, rw_ref, scores_ref, hss_ref, dest_ref, tid_ref, nv_ref):
    hs = hs_ref[...]
    logits = lax.dot_general(hs, rw_ref[...], (((1,), (1,)), ((), ())),
                             preferred_element_type=jnp.float32)
    iota_e = lax.broadcasted_iota(jnp.int32, (T, E), 1)
    mx = jnp.max(logits, axis=1, keepdims=True)
    eid = jnp.min(jnp.where(logits == mx, iota_e, E), axis=1, keepdims=True)
    onehot_b = iota_e == eid
    scores = jnp.where(onehot_b, jax.nn.sigmoid(logits), 0.0)
    scores_ref[...] = scores
    gval = jnp.sum(scores, axis=1, keepdims=True)
    hss_ref[...] = hs * gval

    oh = onehot_b.astype(jnp.bfloat16)
    tt_r = lax.broadcasted_iota(jnp.int32, (T, T), 0)
    tt_c = lax.broadcasted_iota(jnp.int32, (T, T), 1)
    incl = (tt_c <= tt_r).astype(jnp.bfloat16)
    cum = jnp.dot(incl, oh, preferred_element_type=jnp.float32)

    counts = cum[T - 1:T, :]
    padded = jnp.floor((counts + (TILE - 1)) / TILE) * TILE
    ee_r = lax.broadcasted_iota(jnp.int32, (E, E), 0)
    ee_c = lax.broadcasted_iota(jnp.int32, (E, E), 1)
    lt = (ee_r < ee_c).astype(jnp.float32)
    le = (ee_r <= ee_c).astype(jnp.float32)
    start = jnp.dot(padded, lt, preferred_element_type=jnp.float32)
    bound = jnp.dot(padded, le, preferred_element_type=jnp.float32)
    total = bound[:, E - 1:E]

    dest = jnp.sum(oh * (start + cum - 1.0), axis=1, keepdims=True)
    dest_ref[...] = dest.astype(jnp.int32)

    ti = lax.broadcasted_iota(jnp.int32, (NT, 1), 0).astype(jnp.float32) * TILE
    tpos = jnp.minimum(ti, total - TILE)
    tid = jnp.sum((bound <= tpos).astype(jnp.float32), axis=1, keepdims=True)
    tid_ref[...] = tid.astype(jnp.int32)
    nv_ref[...] = (total / TILE).astype(jnp.int32)


def _router_call(hs, router_w, interpret=False):
    return pl.pallas_call(
        _router_body,
        out_shape=[
            jax.ShapeDtypeStruct((T, E), jnp.float32),
            jax.ShapeDtypeStruct((T, H), jnp.float32),
            jax.ShapeDtypeStruct((T, 1), jnp.int32),
            jax.ShapeDtypeStruct((NT, 1), jnp.int32),
            jax.ShapeDtypeStruct((1, 1), jnp.int32),
        ],
        interpret=interpret,
    )(hs, router_w)


def _expert_body(meta_ref, x_ref, g_ref, u_ref, d_ref, o_ref):
    r = pl.program_id(0)
    nv = meta_ref[NT]

    @pl.when(r < nv)
    def _():
        x = x_ref[...].astype(jnp.bfloat16)
        g = jnp.dot(x, g_ref[0].astype(jnp.bfloat16),
                    preferred_element_type=jnp.float32)
        u = jnp.dot(x, u_ref[0].astype(jnp.bfloat16),
                    preferred_element_type=jnp.float32)
        h = ((g * jax.nn.sigmoid(g)) * u).astype(jnp.bfloat16)
        o_ref[...] = jnp.dot(h, d_ref[0].astype(jnp.bfloat16),
                             preferred_element_type=jnp.float32)


def _xm(r, meta):
    nv = meta[NT]
    return (jnp.where(r < nv, r, nv - 1), 0)


def _gm(r, meta):
    return (meta[r], 0, 0)


def _um(r, meta):
    return (meta[r], 0, 1)


def _dm(r, meta):
    return (meta[r], 0, 0)


def _expert_call(meta, xbuf, gate_up_proj, down_proj, interpret=False):
    grid_spec = pltpu.PrefetchScalarGridSpec(
        num_scalar_prefetch=1,
        grid=(NT,),
        in_specs=[
            pl.BlockSpec((TILE, H), _xm),
            pl.BlockSpec((1, H, I), _gm),
            pl.BlockSpec((1, H, I), _um),
            pl.BlockSpec((1, I, H), _dm),
        ],
        out_specs=pl.BlockSpec((TILE, H), _xm),
    )
    return pl.pallas_call(
        _expert_body,
        grid_spec=grid_spec,
        out_shape=jax.ShapeDtypeStruct((P, H), jnp.float32),
        compiler_params=pltpu.CompilerParams(
            dimension_semantics=("arbitrary",)),
        interpret=interpret,
    )(meta, xbuf, gate_up_proj, gate_up_proj, down_proj)


def _shared_body(hs_ref, sg_ref, su_ref, sd_ref, rt_ref, o_ref,
                 sgs_ref, sus_ref, sds_ref):
    r = pl.program_id(0)

    @pl.when(r == 0)
    def _():
        sgs_ref[...] = sg_ref[...].astype(jnp.bfloat16)
        sus_ref[...] = su_ref[...].astype(jnp.bfloat16)
        sds_ref[...] = sd_ref[...].astype(jnp.bfloat16)

    x = hs_ref[...].astype(jnp.bfloat16)
    g = jnp.dot(x, sgs_ref[...], preferred_element_type=jnp.float32)
    u = jnp.dot(x, sus_ref[...], preferred_element_type=jnp.float32)
    h = ((g * jax.nn.sigmoid(g)) * u).astype(jnp.bfloat16)
    o_ref[...] = jnp.dot(h, sds_ref[...],
                         preferred_element_type=jnp.float32) + rt_ref[...]


def _shared_call(hs, shared_gate, shared_up, shared_down, rt, interpret=False):
    return pl.pallas_call(
        _shared_body,
        grid=(T // STILE,),
        in_specs=[
            pl.BlockSpec((STILE, H), lambda r: (r, 0)),
            pl.BlockSpec((H, I), lambda r: (0, 0)),
            pl.BlockSpec((H, I), lambda r: (0, 0)),
            pl.BlockSpec((I, H), lambda r: (0, 0)),
            pl.BlockSpec((STILE, H), lambda r: (r, 0)),
        ],
        out_specs=pl.BlockSpec((STILE, H), lambda r: (r, 0)),
        out_shape=jax.ShapeDtypeStruct((T, H), jnp.float32),
        scratch_shapes=[
            pltpu.VMEM((H, I), jnp.bfloat16),
            pltpu.VMEM((H, I), jnp.bfloat16),
            pltpu.VMEM((I, H), jnp.bfloat16),
        ],
        compiler_params=pltpu.CompilerParams(
            dimension_semantics=("arbitrary",)),
        interpret=interpret,
    )(hs, shared_gate, shared_up, shared_down, rt)


_SC_MESH = dict(core_axis_name="c", subcore_axis_name="s")


def _sc_dispatch_body(hss_hbm, dest_hbm, xbuf_hbm, idx_v, rows_v, sem):
    wid = lax.axis_index("s") * NC + lax.axis_index("c")
    base = wid * TPW
    pltpu.sync_copy(dest_hbm.at[pl.ds(base, TPW)], idx_v)
    pltpu.sync_copy(hss_hbm.at[pl.ds(base, TPW)], rows_v)
    pltpu.async_copy(rows_v, xbuf_hbm.at[idx_v], sem).wait()


def _sc_dispatch(hss, dest):
    return pl.kernel(
        _sc_dispatch_body,
        out_type=jax.ShapeDtypeStruct((P, H), jnp.float32),
        mesh=plsc.VectorSubcoreMesh(**_SC_MESH),
        scratch_types=[
            pltpu.VMEM((TPW,), jnp.int32),
            pltpu.VMEM((TPW, H), jnp.float32),
            pltpu.SemaphoreType.DMA,
        ],
    )(hss, dest)


def _sc_gather_body(routed_hbm, dest_hbm, out_hbm, idx_v, rows_v, sem):
    wid = lax.axis_index("s") * NC + lax.axis_index("c")
    base = wid * TPW
    pltpu.sync_copy(dest_hbm.at[pl.ds(base, TPW)], idx_v)
    pltpu.async_copy(routed_hbm.at[idx_v], rows_v, sem).wait()
    pltpu.sync_copy(rows_v, out_hbm.at[pl.ds(base, TPW)])


def _sc_gather(routed, dest):
    return pl.kernel(
        _sc_gather_body,
        out_type=jax.ShapeDtypeStruct((T, H), jnp.float32),
        mesh=plsc.VectorSubcoreMesh(**_SC_MESH),
        scratch_types=[
            pltpu.VMEM((TPW,), jnp.int32),
            pltpu.VMEM((TPW, H), jnp.float32),
            pltpu.SemaphoreType.DMA,
        ],
    )(routed, dest)


def kernel(hidden_states, router_w, gate_up_proj, down_proj,
           shared_gate, shared_up, shared_down):
    hs = hidden_states.reshape(T, H)
    scores_te, hs_scaled, dest2, tid, nv = _router_call(hs, router_w)
    router_scores = scores_te.T
    dest = dest2.reshape(T)
    meta = jnp.concatenate([tid.reshape(NT), nv.reshape(1)])
    xbuf = _sc_dispatch(hs_scaled, dest)
    routed = _expert_call(meta, xbuf, gate_up_proj, down_proj)
    rt = _sc_gather(routed, dest)
    out = _shared_call(hs, shared_gate, shared_up, shared_down, rt)
    return out, router_scores
